# baseline (speedup 1.0000x reference)
import dataclasses

import jax
import jax.numpy as jnp
from jax import lax
from jax.experimental import pallas as pl
from jax.experimental.pallas import tpu as pltpu
from jax.experimental.pallas import tpu_sc as plsc

N_NODES = 10000
N_REL = 16
D = 128
E = 320000

NC = 2
NS = 16
NW = NC * NS
B = 48
K = 14
G = 15
RPT = K * G
E_PAD = NW * RPT * B
ROWS = E_PAD // B
LANES = 16
N_PAD = 10240
NPT = N_PAD // NS
INV_SQRT_D = float(1.0 / (D ** 0.5))


def _sc_body(ent_hbm, rel_hbm, head_hbm, tail_hbm, type_hbm,
             acc_hbm, den_hbm,
             acc_sh, den_v, headv, tailv, typev,
             h0, h1, t0, t1, msg, rel_t, sem0, sem1):
    c = lax.axis_index("c")
    s = lax.axis_index("s")
    wid = s * NC + c

    zeros16 = jnp.zeros((LANES,), jnp.float32)
    zidx16 = jnp.zeros((LANES,), jnp.int32)
    lane = lax.iota(jnp.int32, LANES)

    @pl.loop(0, B)
    def _(i):
        for j in range(D // LANES):
            msg[i, pl.ds(j * LANES, LANES)] = zeros16

    @pl.loop(0, N_PAD // LANES)
    def _(i):
        den_v[0, pl.ds(i * LANES, LANES)] = zeros16

    nbase = s * NPT

    @pl.loop(0, NPT // B)
    def _(k):
        pltpu.sync_copy(msg, acc_sh.at[pl.ds(nbase + k * B, B)])

    ztail = NPT - (NPT // B) * B
    pltpu.sync_copy(msg.at[pl.ds(0, ztail)],
                    acc_sh.at[pl.ds(nbase + (NPT // B) * B, ztail)])
    pltpu.sync_copy(rel_hbm, rel_t)

    plsc.subcore_barrier()

    def issue_gathers(kb, hb, tb, sem):
        pltpu.async_copy(ent_hbm.at[headv.at[kb, 0]], hb, sem)
        pltpu.async_copy(ent_hbm.at[tailv.at[kb, 0]], tb, sem)

    def wait_gathers(hb, tb, sem):
        pltpu.make_async_copy(ent_hbm.at[pl.ds(0, B)], hb, sem).wait()
        pltpu.make_async_copy(ent_hbm.at[pl.ds(0, B)], tb, sem).wait()

    def compute_batch(kb, hb, tb):
        @plsc.parallel_loop(0, B // LANES)
        def _(g):
            idxv = headv[kb, 0, pl.ds(g * LANES, LANES)]
            tyv = typev[kb, 0, pl.ds(g * LANES, LANES)]
            for l in range(LANES):
                e = g * LANES + l
                ty = tyv[l]
                ks = []
                ps = []
                for j in range(D // LANES):
                    sl = pl.ds(j * LANES, LANES)
                    k = tb[e, sl] * rel_t[ty, sl]
                    ks.append(k)
                    ps.append(hb[e, sl] * k)
                while len(ps) > 1:
                    ps = [a + b for a, b in zip(ps[::2], ps[1::2])]
                sc = jnp.sum(ps[0]) * INV_SQRT_D
                es = jnp.exp(jnp.full((LANES,), sc, jnp.float32))
                for j in range(D // LANES):
                    msg[e, pl.ds(j * LANES, LANES)] = ks[j] * es
                plsc.addupdate_scatter(den_v, [zidx16, idxv], es,
                                       mask=lane == l)

        pltpu.sync_copy(msg, acc_sh.at[headv.at[kb, 0]], add=True)

    @pl.loop(0, G)
    def _(kg):
        rbase = wid * RPT + kg * K
        pltpu.sync_copy(head_hbm.at[pl.ds(rbase, K)], headv)
        pltpu.sync_copy(tail_hbm.at[pl.ds(rbase, K)], tailv)
        pltpu.sync_copy(type_hbm.at[pl.ds(rbase, K)], typev)

        issue_gathers(0, h0, t0, sem0)

        @pl.loop(0, K // 2)
        def _(i):
            kb0 = 2 * i
            kb1 = 2 * i + 1
            issue_gathers(kb1, h1, t1, sem1)
            wait_gathers(h0, t0, sem0)
            compute_batch(kb0, h0, t0)

            @pl.when(i < K // 2 - 1)
            def _():
                issue_gathers(kb0 + 2, h0, t0, sem0)

            wait_gathers(h1, t1, sem1)
            compute_batch(kb1, h1, t1)

    plsc.subcore_barrier()

    @pl.loop(0, NPT // B)
    def _(k):
        sl = pl.ds(nbase + k * B, B)
        pltpu.sync_copy(acc_sh.at[sl], acc_hbm.at[c, sl])

    wtail = NPT - (NPT // B) * B
    sl = pl.ds(nbase + (NPT // B) * B, wtail)
    pltpu.sync_copy(acc_sh.at[sl], acc_hbm.at[c, sl])
    pltpu.sync_copy(den_v, den_hbm.at[wid])


def _finish_body(ent_ref, acc_ref, den_ref, out_ref):
    num = acc_ref[0] + acc_ref[1]
    den = jnp.sum(den_ref[...], axis=1, keepdims=True) + jnp.float32(1e-8)
    out_ref[...] = ent_ref[...] + num / den


def kernel(entity_emb, relation_emb, edge_index, edge_type):
    npad = E_PAD - E
    head2 = jnp.concatenate(
        [edge_index[0], jnp.full((npad,), N_NODES, jnp.int32)]
    ).reshape(ROWS, 1, B)
    tail2 = jnp.concatenate(
        [edge_index[1], jnp.zeros((npad,), jnp.int32)]
    ).reshape(ROWS, 1, B)
    type2 = jnp.concatenate(
        [edge_type, jnp.zeros((npad,), jnp.int32)]
    ).reshape(ROWS, 1, B)

    cp = pltpu.CompilerParams()
    if "needs_layout_passes" in pltpu.CompilerParams.__dataclass_fields__:
        cp = dataclasses.replace(cp, needs_layout_passes=False)

    mesh = plsc.VectorSubcoreMesh(core_axis_name="c", subcore_axis_name="s")
    sc_fn = pl.kernel(
        _sc_body,
        out_type=[
            jax.ShapeDtypeStruct((NC, N_PAD, D), jnp.float32),
            jax.ShapeDtypeStruct((NW, 1, N_PAD), jnp.float32),
        ],
        mesh=mesh,
        scratch_types=[
            pltpu.VMEM_SHARED((N_PAD, D), jnp.float32),
            pltpu.VMEM((1, N_PAD), jnp.float32),
            pltpu.VMEM((K, 1, B), jnp.int32),
            pltpu.VMEM((K, 1, B), jnp.int32),
            pltpu.VMEM((K, 1, B), jnp.int32),
            pltpu.VMEM((B, D), jnp.float32),
            pltpu.VMEM((B, D), jnp.float32),
            pltpu.VMEM((B, D), jnp.float32),
            pltpu.VMEM((B, D), jnp.float32),
            pltpu.VMEM((B, D), jnp.float32),
            pltpu.VMEM((N_REL, D), jnp.float32),
            pltpu.SemaphoreType.DMA,
            pltpu.SemaphoreType.DMA,
        ],
        compiler_params=cp,
    )
    acc, den = sc_fn(entity_emb, relation_emb, head2, tail2, type2)
    den_t = den.reshape(NW, N_PAD).T

    BN = 2000
    out = pl.pallas_call(
        _finish_body,
        out_shape=jax.ShapeDtypeStruct((N_NODES, D), jnp.float32),
        grid=(N_NODES // BN,),
        in_specs=[
            pl.BlockSpec((BN, D), lambda i: (i, 0)),
            pl.BlockSpec((NC, BN, D), lambda i: (0, i, 0)),
            pl.BlockSpec((BN, NW), lambda i: (i, 0)),
        ],
        out_specs=pl.BlockSpec((BN, D), lambda i: (i, 0)),
    )(entity_emb, acc, den_t)
    return out


# speedup vs baseline: 1.0940x; 1.0940x over previous
import dataclasses

import jax
import jax.numpy as jnp
from jax import lax
from jax.experimental import pallas as pl
from jax.experimental.pallas import tpu as pltpu
from jax.experimental.pallas import tpu_sc as plsc

N_NODES = 10000
N_REL = 16
D = 128
E = 320000

NC = 2
NS = 16
NW = NC * NS
B = 40
K = 14
G = 18
RPT = K * G
E_PAD = NW * RPT * B
ROWS = E_PAD // B
LANES = 16
N_PAD = 10240
NPT = N_PAD // NS
INV_SQRT_D = float(1.0 / (D ** 0.5))


def _sc_body(ent_hbm, rel_hbm, ht_hbm, head_hbm, type_hbm,
             acc_hbm, den_hbm,
             acc_sh, den_v, htv, headv, typev,
             g0, g1, msg0, msg1, rel_t, sem0, sem1, ssem0, ssem1):
    c = lax.axis_index("c")
    s = lax.axis_index("s")
    wid = s * NC + c

    zeros16 = jnp.zeros((LANES,), jnp.float32)
    zidx16 = jnp.zeros((LANES,), jnp.int32)
    lane = lax.iota(jnp.int32, LANES)

    @pl.loop(0, B)
    def _(i):
        for j in range(D // LANES):
            msg0[i, pl.ds(j * LANES, LANES)] = zeros16

    @pl.loop(0, N_PAD // LANES)
    def _(i):
        den_v[0, pl.ds(i * LANES, LANES)] = zeros16

    nbase = s * NPT

    @pl.loop(0, NPT // B)
    def _(k):
        pltpu.sync_copy(msg0, acc_sh.at[pl.ds(nbase + k * B, B)])

    pltpu.sync_copy(rel_hbm, rel_t)

    plsc.subcore_barrier()

    def issue_gather(kb, gb, sem):
        pltpu.async_copy(ent_hbm.at[htv.at[kb, 0]], gb, sem)

    def wait_gather(gb, sem):
        pltpu.make_async_copy(ent_hbm.at[pl.ds(0, 2 * B)], gb, sem).wait()

    def issue_scatter(kb, mb, sem):
        pltpu.async_copy(mb, acc_sh.at[headv.at[kb, 0]], sem, add=True)

    def wait_scatter(mb, sem):
        pltpu.make_async_copy(mb, acc_sh.at[pl.ds(0, B)], sem).wait()

    def compute_batch(kb, gb, mb):
        @pl.loop(0, B // LANES)
        def _(g):
            idxv = headv[kb, 0, pl.ds(g * LANES, LANES)]
            tyv = typev[kb, 0, pl.ds(g * LANES, LANES)]
            for l in range(LANES):
                e = g * LANES + l
                ty = tyv[l]
                ks = []
                ps = []
                for j in range(D // LANES):
                    sl = pl.ds(j * LANES, LANES)
                    k = gb[B + e, sl] * rel_t[ty, sl]
                    ks.append(k)
                    ps.append(gb[e, sl] * k)
                while len(ps) > 1:
                    ps = [a + b for a, b in zip(ps[::2], ps[1::2])]
                sc = jnp.sum(ps[0]) * INV_SQRT_D
                es = jnp.exp(jnp.full((LANES,), sc, jnp.float32))
                for j in range(D // LANES):
                    mb[e, pl.ds(j * LANES, LANES)] = ks[j] * es
                plsc.addupdate_scatter(den_v, [zidx16, idxv], es,
                                       mask=lane == l)

    @pl.loop(0, G)
    def _(kg):
        rbase = wid * RPT + kg * K
        pltpu.sync_copy(ht_hbm.at[pl.ds(rbase, K)], htv)
        pltpu.sync_copy(head_hbm.at[pl.ds(rbase, K)], headv)
        pltpu.sync_copy(type_hbm.at[pl.ds(rbase, K)], typev)

        issue_gather(0, g0, sem0)

        @pl.loop(0, K // 2)
        def _(i):
            kb0 = 2 * i
            kb1 = 2 * i + 1
            issue_gather(kb1, g1, sem1)
            wait_gather(g0, sem0)

            @pl.when(i > 0)
            def _():
                wait_scatter(msg0, ssem0)

            compute_batch(kb0, g0, msg0)
            issue_scatter(kb0, msg0, ssem0)

            @pl.when(i < K // 2 - 1)
            def _():
                issue_gather(kb0 + 2, g0, sem0)

            wait_gather(g1, sem1)

            @pl.when(i > 0)
            def _():
                wait_scatter(msg1, ssem1)

            compute_batch(kb1, g1, msg1)
            issue_scatter(kb1, msg1, ssem1)

        wait_scatter(msg0, ssem0)
        wait_scatter(msg1, ssem1)

    plsc.subcore_barrier()

    @pl.loop(0, NPT // B)
    def _(k):
        sl = pl.ds(nbase + k * B, B)
        pltpu.sync_copy(acc_sh.at[sl], acc_hbm.at[c, sl])

    pltpu.sync_copy(den_v, den_hbm.at[wid])


def _finish_body(ent_ref, acc_ref, den_ref, out_ref):
    num = acc_ref[0] + acc_ref[1]
    den = jnp.sum(den_ref[...], axis=1, keepdims=True) + jnp.float32(1e-8)
    out_ref[...] = ent_ref[...] + num / den


def kernel(entity_emb, relation_emb, edge_index, edge_type):
    npad = E_PAD - E
    headR = jnp.concatenate(
        [edge_index[0], jnp.full((npad,), N_NODES, jnp.int32)]
    ).reshape(ROWS, B)
    tailR = jnp.concatenate(
        [edge_index[1], jnp.zeros((npad,), jnp.int32)]
    ).reshape(ROWS, B)
    ht2 = jnp.concatenate([headR, tailR], axis=1).reshape(ROWS, 1, 2 * B)
    head2 = headR.reshape(ROWS, 1, B)
    type2 = jnp.concatenate(
        [edge_type, jnp.zeros((npad,), jnp.int32)]
    ).reshape(ROWS, 1, B)

    cp = pltpu.CompilerParams()
    if "needs_layout_passes" in pltpu.CompilerParams.__dataclass_fields__:
        cp = dataclasses.replace(cp, needs_layout_passes=False)

    mesh = plsc.VectorSubcoreMesh(core_axis_name="c", subcore_axis_name="s")
    sc_fn = pl.kernel(
        _sc_body,
        out_type=[
            jax.ShapeDtypeStruct((NC, N_PAD, D), jnp.float32),
            jax.ShapeDtypeStruct((NW, 1, N_PAD), jnp.float32),
        ],
        mesh=mesh,
        scratch_types=[
            pltpu.VMEM_SHARED((N_PAD, D), jnp.float32),
            pltpu.VMEM((1, N_PAD), jnp.float32),
            pltpu.VMEM((K, 1, 2 * B), jnp.int32),
            pltpu.VMEM((K, 1, B), jnp.int32),
            pltpu.VMEM((K, 1, B), jnp.int32),
            pltpu.VMEM((2 * B, D), jnp.float32),
            pltpu.VMEM((2 * B, D), jnp.float32),
            pltpu.VMEM((B, D), jnp.float32),
            pltpu.VMEM((B, D), jnp.float32),
            pltpu.VMEM((N_REL, D), jnp.float32),
            pltpu.SemaphoreType.DMA,
            pltpu.SemaphoreType.DMA,
            pltpu.SemaphoreType.DMA,
            pltpu.SemaphoreType.DMA,
        ],
        compiler_params=cp,
    )
    acc, den = sc_fn(entity_emb, relation_emb, ht2, head2, type2)
    den_t = den.reshape(NW, N_PAD).T

    BN = 2000
    out = pl.pallas_call(
        _finish_body,
        out_shape=jax.ShapeDtypeStruct((N_NODES, D), jnp.float32),
        grid=(N_NODES // BN,),
        in_specs=[
            pl.BlockSpec((BN, D), lambda i: (i, 0)),
            pl.BlockSpec((NC, BN, D), lambda i: (0, i, 0)),
            pl.BlockSpec((BN, NW), lambda i: (i, 0)),
        ],
        out_specs=pl.BlockSpec((BN, D), lambda i: (i, 0)),
    )(entity_emb, acc, den_t)
    return out
